# Initial kernel scaffold; baseline (speedup 1.0000x reference)
import math

import jax
import jax.numpy as jnp
from jax import lax
from jax.experimental import pallas as pl
from jax.experimental.pallas import tpu as pltpu
from jax.experimental.pallas import tpu_sc as plsc

D = 128
SCALE = math.sqrt(float(D))
L = 16

_info = plsc.get_sparse_core_info()
NC, NS = _info.num_cores, _info.num_subcores
NW = NC * NS

B = 4096 * 50
BPW = B // NW
C = 128
NCH = BPW // C
IDX_ROWS = B // C


def _body(x_hbm, table_hbm, out_hbm, idx_v, g0, g1, sem0, sem1):
    wid = lax.axis_index("s") * NC + lax.axis_index("c")
    idx_row0 = wid * NCH

    pltpu.sync_copy(x_hbm.at[pl.ds(idx_row0, NCH)], idx_v)

    def start_gather(j, buf, sem):
        pltpu.make_async_copy(table_hbm.at[idx_v.at[j]], buf, sem).start()

    def wait_gather(j, buf, sem):
        pltpu.make_async_copy(table_hbm.at[idx_v.at[j]], buf, sem).wait()

    def scale_buf(buf):
        def row(r, carry):
            for c in range(D // L):
                sl = pl.ds(c * L, L)
                buf[r, sl] = buf[r, sl] * SCALE
            return carry
        lax.fori_loop(0, C, row, 0)

    def copy_out(j, buf):
        base = (idx_row0 + j) * C
        pltpu.sync_copy(buf, out_hbm.at[pl.ds(base, C)])

    start_gather(0, g0, sem0)
    start_gather(1, g1, sem1)

    def outer(i, carry):
        j0 = 2 * i
        j1 = j0 + 1

        wait_gather(j0, g0, sem0)
        scale_buf(g0)
        copy_out(j0, g0)

        @pl.when(j0 + 2 < NCH)
        def _():
            start_gather(j0 + 2, g0, sem0)

        wait_gather(j1, g1, sem1)
        scale_buf(g1)
        copy_out(j1, g1)

        @pl.when(j1 + 2 < NCH)
        def _():
            start_gather(j1 + 2, g1, sem1)

        return carry

    lax.fori_loop(0, NCH // 2, outer, 0)


def kernel(x, table):
    xf = x.reshape(-1).astype(jnp.int32).reshape(IDX_ROWS, C)
    mesh = plsc.VectorSubcoreMesh(core_axis_name="c", subcore_axis_name="s")
    out = pl.kernel(
        _body,
        mesh=mesh,
        out_type=jax.ShapeDtypeStruct((B, D), jnp.float32),
        scratch_types=[
            pltpu.VMEM((NCH, C), jnp.int32),
            pltpu.VMEM((C, D), jnp.float32),
            pltpu.VMEM((C, D), jnp.float32),
            pltpu.SemaphoreType.DMA,
            pltpu.SemaphoreType.DMA,
        ],
    )(xf, table)
    return out.reshape(4096, 50, D)


# speedup vs baseline: 2.8875x; 2.8875x over previous
import math

import jax
import jax.numpy as jnp
from jax import lax
from jax.experimental import pallas as pl
from jax.experimental.pallas import tpu as pltpu
from jax.experimental.pallas import tpu_sc as plsc

D = 128
SCALE = math.sqrt(float(D))
L = 16

_info = plsc.get_sparse_core_info()
NC, NS = _info.num_cores, _info.num_subcores
NW = NC * NS

B = 4096 * 50
BPW = B // NW
C = 128
NCH = BPW // C
IDX_ROWS = B // C


def _body(x_hbm, table_hbm, out_hbm, idx_v, g0, g1, sem0, sem1):
    wid = lax.axis_index("s") * NC + lax.axis_index("c")
    idx_row0 = wid * NCH

    pltpu.sync_copy(x_hbm.at[wid], idx_v)

    def start_gather(j, buf, sem):
        pltpu.make_async_copy(table_hbm.at[idx_v.at[j]], buf, sem).start()

    def wait_gather(j, buf, sem):
        pltpu.make_async_copy(table_hbm.at[idx_v.at[j]], buf, sem).wait()

    def scale_buf(buf):
        def row(r, carry):
            for c in range(D // L):
                sl = pl.ds(c * L, L)
                buf[r, sl] = buf[r, sl] * SCALE
            return carry
        lax.fori_loop(0, C, row, 0)

    def copy_out(j, buf):
        base = (idx_row0 + j) * C
        pltpu.sync_copy(buf, out_hbm.at[pl.ds(base, C)])

    start_gather(0, g0, sem0)
    start_gather(1, g1, sem1)

    def outer(i, carry):
        j0 = 2 * i
        j1 = j0 + 1

        wait_gather(j0, g0, sem0)
        scale_buf(g0)
        copy_out(j0, g0)

        @pl.when(j0 + 2 < NCH)
        def _():
            start_gather(j0 + 2, g0, sem0)

        wait_gather(j1, g1, sem1)
        scale_buf(g1)
        copy_out(j1, g1)

        @pl.when(j1 + 2 < NCH)
        def _():
            start_gather(j1 + 2, g1, sem1)

        return carry

    lax.fori_loop(0, NCH // 2, outer, 0)


def kernel(x, table):
    xf = x.reshape(-1).astype(jnp.int32).reshape(NW, NCH, C)
    mesh = plsc.VectorSubcoreMesh(core_axis_name="c", subcore_axis_name="s")
    out = pl.kernel(
        _body,
        mesh=mesh,
        out_type=jax.ShapeDtypeStruct((B, D), jnp.float32),
        scratch_types=[
            pltpu.VMEM((NCH, C), jnp.int32),
            pltpu.VMEM((C, D), jnp.float32),
            pltpu.VMEM((C, D), jnp.float32),
            pltpu.SemaphoreType.DMA,
            pltpu.SemaphoreType.DMA,
        ],
    )(xf, table)
    return out.reshape(4096, 50, D)
